# baseline (speedup 1.0000x reference)
import functools

import jax
import jax.numpy as jnp
import numpy as np
from jax import lax
from jax.experimental import pallas as pl
from jax.experimental.pallas import tpu as pltpu
from jax.experimental.pallas import tpu_sc as plsc

_N = 50000
_D = 128
_B = 16384
_K = 32

_NC = 2
_NS = 16
_NW = _NC * _NS
_RPW = _B // _NW
_R = 4
_CH = _RPW // _R
_L = 16
_NBUF = 8
_DW = _D // 2

_COLS_LO = np.array([32 * w + i for w in range(_D // 32) for i in range(16)],
                    np.int32)
_COLS_HI = _COLS_LO + 16


def _proj_body(emb_ref, wa_ref, wb_ref, wc_ref, wd_ref,
               blo_ref, bhi_ref, p1_ref, p2_ref):
    e = emb_ref[...]
    dn = (((1,), (1,)), ((), ()))

    def hi16(w_ref, scale, bias):
        p = lax.dot_general(e, w_ref[...], dn,
                            preferred_element_type=jnp.float32)
        r = (p * scale + bias).astype(jnp.bfloat16).astype(jnp.float32)
        return lax.shift_right_logical(lax.bitcast_convert_type(r, jnp.int32),
                                       16)

    p1_ref[...] = (lax.shift_left(hi16(wb_ref, 1.0 / _K, 0.0), 16)
                   | hi16(wa_ref, 1.0 / _K, 0.0))
    p2_ref[...] = (lax.shift_left(hi16(wd_ref, 1.0, bhi_ref[...]), 16)
                   | hi16(wc_ref, 1.0, blo_ref[...]))


def _project(embed_matrix, wa, wb, wc, wd, blo, bhi):
    blk = 2000
    grid = (_N // blk,)
    whalf = pl.BlockSpec((_DW, _D), lambda i: (0, 0))
    pout = pl.BlockSpec((blk, _DW), lambda i: (i, 0))
    return pl.pallas_call(
        _proj_body,
        grid=grid,
        in_specs=[
            pl.BlockSpec((blk, _D), lambda i: (i, 0)),
            whalf, whalf, whalf, whalf,
            pl.BlockSpec((1, _DW), lambda i: (0, 0)),
            pl.BlockSpec((1, _DW), lambda i: (0, 0)),
        ],
        out_specs=[pout, pout],
        out_shape=[
            jax.ShapeDtypeStruct((_N, _DW), jnp.int32),
            jax.ShapeDtypeStruct((_N, _DW), jnp.int32),
        ],
    )(embed_matrix, wa, wb, wc, wd, blo, bhi)


def _sc_body(p1, p2, nidx, uidx, out, nidx_v, u_v, nbuf, sbuf, obuf, *sems):
    gsems = sems[:_NBUF]
    osems = sems[_NBUF:]
    wid = lax.axis_index("s") * _NC + lax.axis_index("c")
    base = wid * _RPW
    pltpu.sync_copy(nidx.at[wid], nidx_v)
    pltpu.sync_copy(uidx.at[wid], u_v)

    def gather_parts(c, slot):
        return (
            (p1.at[nidx_v.at[c]], nbuf.at[slot]),
            (p2.at[u_v.at[c]], sbuf.at[slot]),
        )

    def start_gather(c, slot):
        for src, dst in gather_parts(c, slot):
            pltpu.async_copy(src, dst, gsems[slot])

    def wait_gather(c, slot):
        for src, dst in gather_parts(c, slot):
            pltpu.make_async_copy(src, dst, gsems[slot]).wait()

    def out_slice(c):
        return out.at[pl.ds(base + c * _R, _R)]

    for w in range(_NBUF - 1):
        start_gather(w, w)

    mask = jnp.int32(-65536)

    def dec_lo(v):
        return plsc.bitcast(v << 16, jnp.float32)

    def dec_hi(v):
        return plsc.bitcast(v & mask, jnp.float32)

    @pl.loop(0, _CH, step=_NBUF)
    def _ring(g):
        for slot in range(_NBUF):
            c = g + slot
            nxt = c + _NBUF - 1

            @pl.when(nxt < _CH)
            def _():
                start_gather(nxt, (slot + _NBUF - 1) % _NBUF)

            wait_gather(c, slot)

            @pl.when(c >= _NBUF)
            def _():
                pltpu.make_async_copy(obuf.at[slot], out_slice(c - _NBUF),
                                      osems[slot]).wait()

            for r in range(_R):
                for w in range(_D // 32):
                    sv = sbuf[slot, r, pl.ds(16 * w, _L)]
                    acc_lo = dec_lo(sv)
                    acc_hi = dec_hi(sv)
                    for k in range(_K):
                        row = r * _K + k
                        v = nbuf[slot, row, pl.ds(16 * w, _L)]
                        acc_lo = acc_lo + dec_lo(v)
                        acc_hi = acc_hi + dec_hi(v)
                    obuf[slot, r, pl.ds(32 * w, _L)] = acc_lo
                    obuf[slot, r, pl.ds(32 * w + _L, _L)] = acc_hi
            pltpu.async_copy(obuf.at[slot], out_slice(c), osems[slot])

    for slot in range(_NBUF):
        pltpu.make_async_copy(obuf.at[slot], out_slice(_CH - _NBUF + slot),
                              osems[slot]).wait()


_sc_gather = functools.partial(
    pl.kernel,
    out_type=jax.ShapeDtypeStruct((_B, _D), jnp.float32),
    mesh=plsc.VectorSubcoreMesh(core_axis_name="c", subcore_axis_name="s"),
    compiler_params=pltpu.CompilerParams(needs_layout_passes=False,
                                         use_tc_tiling_on_sc=False),
    scratch_types=[
        pltpu.VMEM((_CH, _R * _K), jnp.int32),
        pltpu.VMEM((_CH, _R), jnp.int32),
        pltpu.VMEM((_NBUF, _R * _K, _DW), jnp.int32),
        pltpu.VMEM((_NBUF, _R, _DW), jnp.int32),
        pltpu.VMEM((_NBUF, _R, _D), jnp.float32),
    ] + [pltpu.SemaphoreType.DMA] * (2 * _NBUF),
)(_sc_body)


def kernel(nodes_u, nodes_i, embed_matrix, neigh_idx, W, b):
    del nodes_i
    w1 = W[:, :_D]
    w2 = W[:, _D:]
    wa = w1[_COLS_LO]
    wb = w1[_COLS_HI]
    wc = w2[_COLS_LO]
    wd = w2[_COLS_HI]
    blo = b[_COLS_LO].reshape(1, _DW)
    bhi = b[_COLS_HI].reshape(1, _DW)
    p1, p2 = _project(embed_matrix, wa, wb, wc, wd, blo, bhi)
    nidx = neigh_idx.astype(jnp.int32).reshape(_NW, _CH, _R * _K)
    uidx = nodes_u.astype(jnp.int32).reshape(_NW, _CH, _R)
    return _sc_gather(p1, p2, nidx, uidx)


# speedup vs baseline: 1.0753x; 1.0753x over previous
import functools

import jax
import jax.numpy as jnp
import numpy as np
from jax import lax
from jax.experimental import pallas as pl
from jax.experimental.pallas import tpu as pltpu
from jax.experimental.pallas import tpu_sc as plsc

_N = 50000
_D = 128
_B = 16384
_K = 32

_NC = 2
_NS = 16
_NW = _NC * _NS
_RPW = _B // _NW
_R = 8
_CH = _RPW // _R
_L = 16
_NBUF = 4
_DW = _D // 2

_COLS_LO = np.array([32 * w + i for w in range(_D // 32) for i in range(16)],
                    np.int32)
_COLS_HI = _COLS_LO + 16


def _proj_body(emb_ref, wa_ref, wb_ref, w2_ref, b_ref, p1_ref, p2_ref):
    e = emb_ref[...]
    dn = (((1,), (1,)), ((), ()))

    def hi16(w_ref):
        p = lax.dot_general(e, w_ref[...], dn,
                            preferred_element_type=jnp.float32)
        r = (p * (1.0 / _K)).astype(jnp.bfloat16).astype(jnp.float32)
        return lax.shift_right_logical(lax.bitcast_convert_type(r, jnp.int32),
                                       16)

    p1_ref[...] = lax.shift_left(hi16(wb_ref), 16) | hi16(wa_ref)
    p2 = lax.dot_general(e, w2_ref[...], dn,
                         preferred_element_type=jnp.float32)
    p2_ref[...] = p2 + b_ref[...]


def _project(embed_matrix, wa, wb, w2, b2d):
    blk = 1000
    grid = (_N // blk,)
    return pl.pallas_call(
        _proj_body,
        grid=grid,
        in_specs=[
            pl.BlockSpec((blk, _D), lambda i: (i, 0)),
            pl.BlockSpec((_DW, _D), lambda i: (0, 0)),
            pl.BlockSpec((_DW, _D), lambda i: (0, 0)),
            pl.BlockSpec((_D, _D), lambda i: (0, 0)),
            pl.BlockSpec((1, _D), lambda i: (0, 0)),
        ],
        out_specs=[
            pl.BlockSpec((blk, _DW), lambda i: (i, 0)),
            pl.BlockSpec((blk, _D), lambda i: (i, 0)),
        ],
        out_shape=[
            jax.ShapeDtypeStruct((_N, _DW), jnp.int32),
            jax.ShapeDtypeStruct((_N, _D), jnp.float32),
        ],
    )(embed_matrix, wa, wb, w2, b2d)


def _sc_body(p1, p2, nidx, uidx, out, nidx_v, u_v, nbuf, sbuf, obuf, *sems):
    gsems = sems[:_NBUF]
    osems = sems[_NBUF:]
    wid = lax.axis_index("s") * _NC + lax.axis_index("c")
    base = wid * _RPW
    pltpu.sync_copy(nidx.at[wid], nidx_v)
    pltpu.sync_copy(uidx.at[wid], u_v)

    def gather_parts(c, slot):
        half = _R * _K // 2
        return (
            (p1.at[nidx_v.at[c, 0]], nbuf.at[slot, pl.ds(0, half)]),
            (p1.at[nidx_v.at[c, 1]], nbuf.at[slot, pl.ds(half, half)]),
            (p2.at[u_v.at[c]], sbuf.at[slot]),
        )

    def start_gather(c, slot):
        for src, dst in gather_parts(c, slot):
            pltpu.async_copy(src, dst, gsems[slot])

    def wait_gather(c, slot):
        for src, dst in gather_parts(c, slot):
            pltpu.make_async_copy(src, dst, gsems[slot]).wait()

    def out_slice(c):
        return out.at[pl.ds(base + c * _R, _R)]

    for w in range(_NBUF - 1):
        start_gather(w, w)

    mask = jnp.int32(-65536)

    @pl.loop(0, _CH, step=_NBUF)
    def _ring(g):
        for slot in range(_NBUF):
            c = g + slot
            nxt = c + _NBUF - 1

            @pl.when(nxt < _CH)
            def _():
                start_gather(nxt, (slot + _NBUF - 1) % _NBUF)

            wait_gather(c, slot)

            @pl.when(c >= _NBUF)
            def _():
                pltpu.make_async_copy(obuf.at[slot], out_slice(c - _NBUF),
                                      osems[slot]).wait()

            for r in range(_R):
                for w in range(_D // 32):
                    acc_lo = sbuf[slot, r, pl.ds(32 * w, _L)]
                    acc_hi = sbuf[slot, r, pl.ds(32 * w + _L, _L)]
                    for k in range(_K):
                        v = nbuf[slot, r * _K + k, pl.ds(16 * w, _L)]
                        acc_lo = acc_lo + plsc.bitcast(v << 16, jnp.float32)
                        acc_hi = acc_hi + plsc.bitcast(v & mask, jnp.float32)
                    obuf[slot, r, pl.ds(32 * w, _L)] = acc_lo
                    obuf[slot, r, pl.ds(32 * w + _L, _L)] = acc_hi
            pltpu.async_copy(obuf.at[slot], out_slice(c), osems[slot])

    for slot in range(_NBUF):
        pltpu.make_async_copy(obuf.at[slot], out_slice(_CH - _NBUF + slot),
                              osems[slot]).wait()


_sc_gather = functools.partial(
    pl.kernel,
    out_type=jax.ShapeDtypeStruct((_B, _D), jnp.float32),
    mesh=plsc.VectorSubcoreMesh(core_axis_name="c", subcore_axis_name="s"),
    compiler_params=pltpu.CompilerParams(needs_layout_passes=False,
                                         use_tc_tiling_on_sc=False),
    scratch_types=[
        pltpu.VMEM((_CH, 2, _R * _K // 2), jnp.int32),
        pltpu.VMEM((_CH, _R), jnp.int32),
        pltpu.VMEM((_NBUF, _R * _K, _DW), jnp.int32),
        pltpu.VMEM((_NBUF, _R, _D), jnp.float32),
        pltpu.VMEM((_NBUF, _R, _D), jnp.float32),
    ] + [pltpu.SemaphoreType.DMA] * (2 * _NBUF),
)(_sc_body)


def kernel(nodes_u, nodes_i, embed_matrix, neigh_idx, W, b):
    del nodes_i
    w1 = W[:, :_D]
    wa = w1[_COLS_LO]
    wb = w1[_COLS_HI]
    w2 = W[:, _D:]
    p1, p2 = _project(embed_matrix, wa, wb, w2, b.reshape(1, _D))
    nidx = neigh_idx.astype(jnp.int32).reshape(_NW, _CH, 2, _R * _K // 2)
    uidx = nodes_u.astype(jnp.int32).reshape(_NW, _CH, _R)
    return _sc_gather(p1, p2, nidx, uidx)
